# Initial kernel scaffold; baseline (speedup 1.0000x reference)
import jax
import jax.numpy as jnp
from jax.experimental import pallas as pl


def kernel(obs, memory, W, b):
    raise NotImplementedError("write your pallas kernel here")


# speedup vs baseline: 2.0562x; 2.0562x over previous
import jax
import jax.numpy as jnp
from jax import lax
from jax.experimental import pallas as pl
from jax.experimental.pallas import tpu as pltpu

_Q = 32
_D = 512
_BM = 5000
_K = 5


def _knn_kernel(obs_ref, W_ref, b_ref, mem_ref, out_ref, emb_s, run_s):
    i = pl.program_id(0)
    nb = pl.num_programs(0)

    @pl.when(i == 0)
    def _init():
        emb = lax.dot_general(
            obs_ref[...], W_ref[...], (((1,), (0,)), ((), ())),
            preferred_element_type=jnp.float32)
        emb_s[...] = emb + b_ref[...]
        run_s[...] = jnp.full((_Q, 128), jnp.inf, jnp.float32)

    mem = mem_ref[...]
    emb = emb_s[...]
    s = lax.dot_general(emb, mem, (((1,), (1,)), ((), ())),
                        preferred_element_type=jnp.float32)
    msq = mem * mem
    ones = jnp.ones((8, _D), jnp.float32)
    m2 = lax.dot_general(ones, msq, (((1,), (1,)), ((), ())),
                         preferred_element_type=jnp.float32)
    t = m2[0:1, :] - 2.0 * s

    v = jnp.concatenate([run_s[...], t], axis=1)
    iota = lax.broadcasted_iota(jnp.int32, v.shape, 1)
    liota = lax.broadcasted_iota(jnp.int32, (_Q, 128), 1)
    newrun = jnp.full((_Q, 128), jnp.inf, jnp.float32)
    for k in range(_K):
        m = jnp.min(v, axis=1, keepdims=True)
        ismin = v == m
        fidx = jnp.min(jnp.where(ismin, iota, jnp.iinfo(jnp.int32).max),
                       axis=1, keepdims=True)
        v = jnp.where(iota == fidx, jnp.inf, v)
        newrun = jnp.where(liota == k, m, newrun)
    run_s[...] = newrun

    @pl.when(i == nb - 1)
    def _fin():
        e = emb_s[...]
        q2 = jnp.sum(e * e, axis=1, keepdims=True)
        d2 = jnp.maximum(run_s[...] + q2, 0.0) + 1e-12
        dist = jnp.sqrt(d2)
        out_ref[0, 0] = jnp.sum(jnp.where(liota < _K, dist, 0.0)) / (_Q * _K)


def kernel(obs, memory, W, b):
    nb = memory.shape[0] // _BM
    b2 = b.reshape(1, _D)
    out = pl.pallas_call(
        _knn_kernel,
        grid=(nb,),
        in_specs=[
            pl.BlockSpec(obs.shape, lambda i: (0, 0)),
            pl.BlockSpec(W.shape, lambda i: (0, 0)),
            pl.BlockSpec((1, _D), lambda i: (0, 0)),
            pl.BlockSpec((_BM, _D), lambda i: (i, 0)),
        ],
        out_specs=pl.BlockSpec((1, 1), lambda i: (0, 0),
                               memory_space=pltpu.SMEM),
        out_shape=jax.ShapeDtypeStruct((1, 1), jnp.float32),
        scratch_shapes=[
            pltpu.VMEM((_Q, _D), jnp.float32),
            pltpu.VMEM((_Q, 128), jnp.float32),
        ],
    )(obs, W, b2, memory)
    return out[0, 0]
